# baseline (speedup 1.0000x reference)
import functools

import jax
import jax.numpy as jnp
from jax import lax
from jax.experimental import pallas as pl
from jax.experimental.pallas import tpu as pltpu
from jax.experimental.pallas import tpu_sc as plsc

B, N, S = 16, 1024, 4096
D1, D2 = 256, 256
BS = B * S
EPS = 1e-5

SB = 512


def _k1_body(x1_ref, x2_ref, idx_ref, w_ref):
    b = pl.program_id(0)
    x1 = x1_ref[0]
    x2 = x2_ref[0]
    x1a, x1b, x1c = x1[0:1, :], x1[1:2, :], x1[2:3, :]
    x2a, x2b, x2c = x2[:, 0:1], x2[:, 1:2], x2[:, 2:3]
    sq1 = x1a * x1a + x1b * x1b + x1c * x1c
    sq2 = x2a * x2a + x2b * x2b + x2c * x2c
    cross = lax.dot_general(x2, x1, (((1,), (0,)), ((), ())),
                            preferred_element_type=jnp.float32)
    d = -2.0 * cross + sq2 + sq1
    lane = lax.broadcasted_iota(jnp.int32, (SB, N), 1)
    inf = jnp.float32(jnp.inf)

    big = jnp.int32(1 << 30)
    m1 = jnp.min(d, axis=1, keepdims=True)
    e1 = d == m1
    d2 = jnp.where(e1, inf, d)
    m2 = jnp.min(d2, axis=1, keepdims=True)
    e2 = d2 == m2
    d3 = jnp.where(e2, inf, d2)
    m3 = jnp.min(d3, axis=1, keepdims=True)
    e3 = d3 == m3
    i1 = jnp.min(jnp.where(e1, lane, big), axis=1, keepdims=True)
    i2 = jnp.min(jnp.where(e2, lane, big), axis=1, keepdims=True)
    i3 = jnp.min(jnp.where(e3, lane, big), axis=1, keepdims=True)

    r1 = 1.0 / (m1 + 1e-8)
    r2 = 1.0 / (m2 + 1e-8)
    r3 = 1.0 / (m3 + 1e-8)
    nrm = r1 + r2 + r3
    idx_ref[0] = jnp.concatenate([i1, i2, i3], axis=1) + b * N
    w_ref[0] = jnp.concatenate([r1, r2, r3], axis=1) / nrm


def _top3(xyz1, x2t):
    return pl.pallas_call(
        _k1_body,
        grid=(B, S // SB),
        in_specs=[
            pl.BlockSpec((1, 3, N), lambda b, j: (b, 0, 0)),
            pl.BlockSpec((1, SB, 3), lambda b, j: (b, j, 0)),
        ],
        out_specs=[
            pl.BlockSpec((1, SB, 3), lambda b, j: (b, j, 0)),
            pl.BlockSpec((1, SB, 3), lambda b, j: (b, j, 0)),
        ],
        out_shape=[
            jax.ShapeDtypeStruct((B, S, 3), jnp.int32),
            jax.ShapeDtypeStruct((B, S, 3), jnp.float32),
        ],
    )(xyz1, x2t)


NW = 32
CH = BS // NW
T = 64
TS = CH // T


def _sc_body(idx0, idx1, idx2, w0, w1, w2, p1t, out_hbm,
             i0v, i1v, i2v, w0v, w1v, w2v, b0, b1, b2, ob, sem):
    c = lax.axis_index("c")
    s = lax.axis_index("s")
    wid = s * 2 + c
    base = pl.multiple_of(wid * CH, CH)
    pltpu.sync_copy(idx0.at[pl.ds(base, CH)], i0v)
    pltpu.sync_copy(idx1.at[pl.ds(base, CH)], i1v)
    pltpu.sync_copy(idx2.at[pl.ds(base, CH)], i2v)
    pltpu.sync_copy(w0.at[pl.ds(base, CH)], w0v)
    pltpu.sync_copy(w1.at[pl.ds(base, CH)], w1v)
    pltpu.sync_copy(w2.at[pl.ds(base, CH)], w2v)

    def step(t, carry):
        row0 = pl.multiple_of(t * T, T)
        cp0 = pltpu.async_copy(p1t.at[i0v.at[pl.ds(row0, T)]], b0, sem)
        cp1 = pltpu.async_copy(p1t.at[i1v.at[pl.ds(row0, T)]], b1, sem)
        cp2 = pltpu.async_copy(p1t.at[i2v.at[pl.ds(row0, T)]], b2, sem)
        cp0.wait()
        cp1.wait()
        cp2.wait()

        def grpfn(g, carry2):
            gbase = row0 + g * 16
            wv0 = w0v[pl.ds(gbase, 16)]
            wv1 = w1v[pl.ds(gbase, 16)]
            wv2 = w2v[pl.ds(gbase, 16)]
            for r16 in range(16):
                r = g * 16 + r16
                s0 = wv0[r16]
                s1 = wv1[r16]
                s2 = wv2[r16]
                for cc in range(D1 // 16):
                    sl = pl.ds(cc * 16, 16)
                    ob[r, sl] = (s0 * b0[r, sl] + s1 * b1[r, sl]
                                 + s2 * b2[r, sl])
            return carry2

        lax.fori_loop(0, T // 16, grpfn, 0)
        pltpu.sync_copy(ob, out_hbm.at[pl.ds(base + row0, T)])
        return carry

    lax.fori_loop(0, TS, step, 0)


def _interp_sc(idx0, idx1, idx2, w0, w1, w2, p1t):
    mesh = plsc.VectorSubcoreMesh(core_axis_name="c", subcore_axis_name="s")
    f = functools.partial(
        pl.kernel,
        mesh=mesh,
        out_type=jax.ShapeDtypeStruct((BS, D1), jnp.float32),
        scratch_types=[
            pltpu.VMEM((CH,), jnp.int32),
            pltpu.VMEM((CH,), jnp.int32),
            pltpu.VMEM((CH,), jnp.int32),
            pltpu.VMEM((CH,), jnp.float32),
            pltpu.VMEM((CH,), jnp.float32),
            pltpu.VMEM((CH,), jnp.float32),
            pltpu.VMEM((T, D1), jnp.float32),
            pltpu.VMEM((T, D1), jnp.float32),
            pltpu.VMEM((T, D1), jnp.float32),
            pltpu.VMEM((T, D1), jnp.float32),
            pltpu.SemaphoreType.DMA,
        ],
    )(_sc_body)
    return f(idx0, idx1, idx2, w0, w1, w2, p1t)


BS3 = 1024
G3 = BS // BS3


def _k3_body(p2_ref, it_ref, wa_ref, wb_ref, b1_ref, h_ref, sm_ref, sq_ref, acc):
    i = pl.program_id(0)

    @pl.when(i == 0)
    def _():
        acc[...] = jnp.zeros_like(acc)

    p2b = p2_ref[0]
    ib = it_ref[...]
    bf = jnp.bfloat16
    t1 = lax.dot_general(p2b.astype(bf), wa_ref[...].astype(bf),
                         (((0,), (0,)), ((), ())),
                         preferred_element_type=jnp.float32)
    t2 = lax.dot_general(ib.astype(bf), wb_ref[...].astype(bf),
                         (((1,), (0,)), ((), ())),
                         preferred_element_type=jnp.float32)
    h = t1 + t2 + b1_ref[...]
    h_ref[...] = h
    acc[0:1] += jnp.sum(h, axis=0, keepdims=True)
    acc[1:2] += jnp.sum(h * h, axis=0, keepdims=True)

    @pl.when(i == G3 - 1)
    def _():
        sm_ref[...] = acc[0:1]
        sq_ref[...] = acc[1:2]


def _layer1(p2, interp, w1at, w1bt, b1r):
    return pl.pallas_call(
        _k3_body,
        grid=(G3,),
        in_specs=[
            pl.BlockSpec((1, D2, BS3), lambda i: (i // (S // BS3), 0, i % (S // BS3))),
            pl.BlockSpec((BS3, D1), lambda i: (i, 0)),
            pl.BlockSpec((D2, D1), lambda i: (0, 0)),
            pl.BlockSpec((D1, D1), lambda i: (0, 0)),
            pl.BlockSpec((1, D1), lambda i: (0, 0)),
        ],
        out_specs=[
            pl.BlockSpec((BS3, D1), lambda i: (i, 0)),
            pl.BlockSpec((1, D1), lambda i: (0, 0)),
            pl.BlockSpec((1, D1), lambda i: (0, 0)),
        ],
        out_shape=[
            jax.ShapeDtypeStruct((BS, D1), jnp.float32),
            jax.ShapeDtypeStruct((1, D1), jnp.float32),
            jax.ShapeDtypeStruct((1, D1), jnp.float32),
        ],
        scratch_shapes=[pltpu.VMEM((2, D1), jnp.float32)],
    )(p2, interp, w1at, w1bt, b1r)


def _k4_body(h_ref, sm1_ref, sq1_ref, g1_ref, be1_ref, w2_ref, b2_ref,
             o_ref, sm_ref, sq_ref, acc):
    i = pl.program_id(0)

    @pl.when(i == 0)
    def _():
        acc[...] = jnp.zeros_like(acc)

    n = jnp.float32(BS)
    mean = sm1_ref[...] / n
    var = sq1_ref[...] / n - mean * mean
    scale = g1_ref[...] * lax.rsqrt(var + EPS)
    a = jnp.maximum((h_ref[...] - mean) * scale + be1_ref[...], 0.0)
    bf = jnp.bfloat16
    h2 = lax.dot_general(a.astype(bf), w2_ref[...].astype(bf),
                         (((1,), (0,)), ((), ())),
                         preferred_element_type=jnp.float32) + b2_ref[...]
    o_ref[...] = h2
    acc[0:1] += jnp.sum(h2, axis=0, keepdims=True)
    acc[1:2] += jnp.sum(h2 * h2, axis=0, keepdims=True)

    @pl.when(i == G3 - 1)
    def _():
        sm_ref[...] = acc[0:1]
        sq_ref[...] = acc[1:2]


def _layer2(h1, sm1, sq1, g1, be1, w2t, b2r):
    return pl.pallas_call(
        _k4_body,
        grid=(G3,),
        in_specs=[
            pl.BlockSpec((BS3, D1), lambda i: (i, 0)),
            pl.BlockSpec((1, D1), lambda i: (0, 0)),
            pl.BlockSpec((1, D1), lambda i: (0, 0)),
            pl.BlockSpec((1, D1), lambda i: (0, 0)),
            pl.BlockSpec((1, D1), lambda i: (0, 0)),
            pl.BlockSpec((D1, D1), lambda i: (0, 0)),
            pl.BlockSpec((1, D1), lambda i: (0, 0)),
        ],
        out_specs=[
            pl.BlockSpec((BS3, D1), lambda i: (i, 0)),
            pl.BlockSpec((1, D1), lambda i: (0, 0)),
            pl.BlockSpec((1, D1), lambda i: (0, 0)),
        ],
        out_shape=[
            jax.ShapeDtypeStruct((BS, D1), jnp.float32),
            jax.ShapeDtypeStruct((1, D1), jnp.float32),
            jax.ShapeDtypeStruct((1, D1), jnp.float32),
        ],
        scratch_shapes=[pltpu.VMEM((2, D1), jnp.float32)],
    )(h1, sm1, sq1, g1, be1, w2t, b2r)


def _k5_body(h_ref, sm_ref, sq_ref, g_ref, be_ref, o_ref):
    n = jnp.float32(BS)
    mean = sm_ref[...] / n
    var = sq_ref[...] / n - mean * mean
    scale = g_ref[...] * lax.rsqrt(var + EPS)
    o_ref[...] = jnp.maximum((h_ref[...] - mean) * scale + be_ref[...], 0.0)


def _bn2(h2, sm2, sq2, g2, be2):
    return pl.pallas_call(
        _k5_body,
        grid=(G3,),
        in_specs=[
            pl.BlockSpec((BS3, D1), lambda i: (i, 0)),
            pl.BlockSpec((1, D1), lambda i: (0, 0)),
            pl.BlockSpec((1, D1), lambda i: (0, 0)),
            pl.BlockSpec((1, D1), lambda i: (0, 0)),
            pl.BlockSpec((1, D1), lambda i: (0, 0)),
        ],
        out_specs=pl.BlockSpec((BS3, D1), lambda i: (i, 0)),
        out_shape=jax.ShapeDtypeStruct((BS, D1), jnp.float32),
    )(h2, sm2, sq2, g2, be2)


@jax.jit
def kernel(xyz1, xyz2, points1, points2, W1, b1, gamma1, beta1,
           W2, b2, gamma2, beta2):
    x2t = jnp.transpose(xyz2, (0, 2, 1))
    idx, w = _top3(xyz1, x2t)

    p1t = jnp.transpose(points1, (0, 2, 1)).reshape(BS // 4, D1)

    idxr = idx.reshape(BS, 3)
    wr = w.reshape(BS, 3)
    interp = _interp_sc(idxr[:, 0], idxr[:, 1], idxr[:, 2],
                        wr[:, 0], wr[:, 1], wr[:, 2], p1t)

    w1at = jnp.transpose(W1[:, :D2])
    w1bt = jnp.transpose(W1[:, D2:])
    h1, sm1, sq1 = _layer1(points2, interp, w1at, w1bt, b1.reshape(1, -1))
    h2, sm2, sq2 = _layer2(h1, sm1, sq1, gamma1.reshape(1, -1),
                           beta1.reshape(1, -1), jnp.transpose(W2),
                           b2.reshape(1, -1))
    out = _bn2(h2, sm2, sq2, gamma2.reshape(1, -1), beta2.reshape(1, -1))
    return jnp.transpose(out.reshape(B, S, D1), (0, 2, 1))


# speedup vs baseline: 1.0427x; 1.0427x over previous
import functools

import jax
import jax.numpy as jnp
from jax import lax
from jax.experimental import pallas as pl
from jax.experimental.pallas import tpu as pltpu
from jax.experimental.pallas import tpu_sc as plsc

B, N, S = 16, 1024, 4096
D1, D2 = 256, 256
BS = B * S
EPS = 1e-5

SB = 512


def _k1_body(x1_ref, x2_ref, idx_ref, w_ref):
    b = pl.program_id(0)
    x1 = x1_ref[0]
    x2 = x2_ref[0]
    x1a, x1b, x1c = x1[0:1, :], x1[1:2, :], x1[2:3, :]
    x2a, x2b, x2c = x2[:, 0:1], x2[:, 1:2], x2[:, 2:3]
    sq1 = x1a * x1a + x1b * x1b + x1c * x1c
    sq2 = x2a * x2a + x2b * x2b + x2c * x2c
    cross = lax.dot_general(x2, x1, (((1,), (0,)), ((), ())),
                            preferred_element_type=jnp.float32)
    d = -2.0 * cross + sq2 + sq1
    lane = lax.broadcasted_iota(jnp.int32, (SB, N), 1)
    inf = jnp.float32(jnp.inf)

    big = jnp.int32(1 << 30)
    m1 = jnp.min(d, axis=1, keepdims=True)
    e1 = d == m1
    d2 = jnp.where(e1, inf, d)
    m2 = jnp.min(d2, axis=1, keepdims=True)
    e2 = d2 == m2
    d3 = jnp.where(e2, inf, d2)
    m3 = jnp.min(d3, axis=1, keepdims=True)
    e3 = d3 == m3
    i1 = jnp.min(jnp.where(e1, lane, big), axis=1, keepdims=True)
    i2 = jnp.min(jnp.where(e2, lane, big), axis=1, keepdims=True)
    i3 = jnp.min(jnp.where(e3, lane, big), axis=1, keepdims=True)

    r1 = 1.0 / (m1 + 1e-8)
    r2 = 1.0 / (m2 + 1e-8)
    r3 = 1.0 / (m3 + 1e-8)
    nrm = r1 + r2 + r3
    idx_ref[0] = jnp.concatenate([i1, i2, i3], axis=1) + b * N
    w_ref[0] = jnp.concatenate([r1, r2, r3], axis=1) / nrm


def _top3(xyz1, x2t):
    return pl.pallas_call(
        _k1_body,
        grid=(B, S // SB),
        in_specs=[
            pl.BlockSpec((1, 3, N), lambda b, j: (b, 0, 0)),
            pl.BlockSpec((1, SB, 3), lambda b, j: (b, j, 0)),
        ],
        out_specs=[
            pl.BlockSpec((1, SB, 3), lambda b, j: (b, j, 0)),
            pl.BlockSpec((1, SB, 3), lambda b, j: (b, j, 0)),
        ],
        out_shape=[
            jax.ShapeDtypeStruct((B, S, 3), jnp.int32),
            jax.ShapeDtypeStruct((B, S, 3), jnp.float32),
        ],
    )(xyz1, x2t)


NW = 32
CH = BS // NW
T = 64
TS = CH // T


def _sc_body(idx0, idx1, idx2, w0, w1, w2, p1t, out_hbm,
             i0v, i1v, i2v, w0v, w1v, w2v, b0, b1, b2, ob, sem):
    c = lax.axis_index("c")
    s = lax.axis_index("s")
    wid = s * 2 + c
    base = pl.multiple_of(wid * CH, CH)
    pltpu.sync_copy(idx0.at[pl.ds(base, CH)], i0v)
    pltpu.sync_copy(idx1.at[pl.ds(base, CH)], i1v)
    pltpu.sync_copy(idx2.at[pl.ds(base, CH)], i2v)
    pltpu.sync_copy(w0.at[pl.ds(base, CH)], w0v)
    pltpu.sync_copy(w1.at[pl.ds(base, CH)], w1v)
    pltpu.sync_copy(w2.at[pl.ds(base, CH)], w2v)

    def step(t, carry):
        row0 = pl.multiple_of(t * T, T)
        cp0 = pltpu.async_copy(p1t.at[i0v.at[pl.ds(row0, T)]], b0, sem)
        cp1 = pltpu.async_copy(p1t.at[i1v.at[pl.ds(row0, T)]], b1, sem)
        cp2 = pltpu.async_copy(p1t.at[i2v.at[pl.ds(row0, T)]], b2, sem)
        cp0.wait()
        cp1.wait()
        cp2.wait()

        def grpfn(g, carry2):
            gbase = row0 + g * 16
            wv0 = w0v[pl.ds(gbase, 16)]
            wv1 = w1v[pl.ds(gbase, 16)]
            wv2 = w2v[pl.ds(gbase, 16)]
            for r16 in range(16):
                r = g * 16 + r16
                s0 = wv0[r16]
                s1 = wv1[r16]
                s2 = wv2[r16]
                for cc in range(D1 // 16):
                    sl = pl.ds(cc * 16, 16)
                    ob[r, sl] = (s0 * b0[r, sl] + s1 * b1[r, sl]
                                 + s2 * b2[r, sl])
            return carry2

        lax.fori_loop(0, T // 16, grpfn, 0)
        pltpu.sync_copy(ob, out_hbm.at[pl.ds(base + row0, T)])
        return carry

    lax.fori_loop(0, TS, step, 0)


def _interp_sc(idx0, idx1, idx2, w0, w1, w2, p1t):
    mesh = plsc.VectorSubcoreMesh(core_axis_name="c", subcore_axis_name="s")
    f = functools.partial(
        pl.kernel,
        mesh=mesh,
        out_type=jax.ShapeDtypeStruct((BS, D1), jnp.float32),
        scratch_types=[
            pltpu.VMEM((CH,), jnp.int32),
            pltpu.VMEM((CH,), jnp.int32),
            pltpu.VMEM((CH,), jnp.int32),
            pltpu.VMEM((CH,), jnp.float32),
            pltpu.VMEM((CH,), jnp.float32),
            pltpu.VMEM((CH,), jnp.float32),
            pltpu.VMEM((T, D1), jnp.float32),
            pltpu.VMEM((T, D1), jnp.float32),
            pltpu.VMEM((T, D1), jnp.float32),
            pltpu.VMEM((T, D1), jnp.float32),
            pltpu.SemaphoreType.DMA,
        ],
    )(_sc_body)
    return f(idx0, idx1, idx2, w0, w1, w2, p1t)


BS3 = 1024
G3 = BS // BS3


def _k3_body(p2_ref, it_ref, wa_ref, wb_ref, b1_ref, h_ref, sm_ref, sq_ref, acc):
    i = pl.program_id(0)

    @pl.when(i == 0)
    def _():
        acc[...] = jnp.zeros_like(acc)

    p2b = p2_ref[0]
    ib = it_ref[...]
    bf = jnp.bfloat16
    t1 = lax.dot_general(p2b.astype(bf), wa_ref[...].astype(bf),
                         (((0,), (0,)), ((), ())),
                         preferred_element_type=jnp.float32)
    t2 = lax.dot_general(ib.astype(bf), wb_ref[...].astype(bf),
                         (((1,), (0,)), ((), ())),
                         preferred_element_type=jnp.float32)
    h = t1 + t2 + b1_ref[...]
    h_ref[...] = h
    acc[0:1] += jnp.sum(h, axis=0, keepdims=True)
    acc[1:2] += jnp.sum(h * h, axis=0, keepdims=True)

    @pl.when(i == G3 - 1)
    def _():
        sm_ref[...] = acc[0:1]
        sq_ref[...] = acc[1:2]


def _layer1(p2, interp, w1at, w1bt, b1r):
    return pl.pallas_call(
        _k3_body,
        grid=(G3,),
        in_specs=[
            pl.BlockSpec((1, D2, BS3), lambda i: (i // (S // BS3), 0, i % (S // BS3))),
            pl.BlockSpec((BS3, D1), lambda i: (i, 0)),
            pl.BlockSpec((D2, D1), lambda i: (0, 0)),
            pl.BlockSpec((D1, D1), lambda i: (0, 0)),
            pl.BlockSpec((1, D1), lambda i: (0, 0)),
        ],
        out_specs=[
            pl.BlockSpec((BS3, D1), lambda i: (i, 0)),
            pl.BlockSpec((1, D1), lambda i: (0, 0)),
            pl.BlockSpec((1, D1), lambda i: (0, 0)),
        ],
        out_shape=[
            jax.ShapeDtypeStruct((BS, D1), jnp.float32),
            jax.ShapeDtypeStruct((1, D1), jnp.float32),
            jax.ShapeDtypeStruct((1, D1), jnp.float32),
        ],
        scratch_shapes=[pltpu.VMEM((2, D1), jnp.float32)],
    )(p2, interp, w1at, w1bt, b1r)


def _k4_body(h_ref, sm1_ref, sq1_ref, g1_ref, be1_ref, w2_ref, b2_ref,
             o_ref, sm_ref, sq_ref, acc):
    i = pl.program_id(0)

    @pl.when(i == 0)
    def _():
        acc[...] = jnp.zeros_like(acc)

    n = jnp.float32(BS)
    mean = sm1_ref[...] / n
    var = sq1_ref[...] / n - mean * mean
    scale = g1_ref[...] * lax.rsqrt(var + EPS)
    a = jnp.maximum((h_ref[...] - mean) * scale + be1_ref[...], 0.0)
    bf = jnp.bfloat16
    h2 = lax.dot_general(a.astype(bf), w2_ref[...].astype(bf),
                         (((1,), (0,)), ((), ())),
                         preferred_element_type=jnp.float32) + b2_ref[...]
    o_ref[...] = h2
    acc[0:1] += jnp.sum(h2, axis=0, keepdims=True)
    acc[1:2] += jnp.sum(h2 * h2, axis=0, keepdims=True)

    @pl.when(i == G3 - 1)
    def _():
        sm_ref[...] = acc[0:1]
        sq_ref[...] = acc[1:2]


def _layer2(h1, sm1, sq1, g1, be1, w2t, b2r):
    return pl.pallas_call(
        _k4_body,
        grid=(G3,),
        in_specs=[
            pl.BlockSpec((BS3, D1), lambda i: (i, 0)),
            pl.BlockSpec((1, D1), lambda i: (0, 0)),
            pl.BlockSpec((1, D1), lambda i: (0, 0)),
            pl.BlockSpec((1, D1), lambda i: (0, 0)),
            pl.BlockSpec((1, D1), lambda i: (0, 0)),
            pl.BlockSpec((D1, D1), lambda i: (0, 0)),
            pl.BlockSpec((1, D1), lambda i: (0, 0)),
        ],
        out_specs=[
            pl.BlockSpec((BS3, D1), lambda i: (i, 0)),
            pl.BlockSpec((1, D1), lambda i: (0, 0)),
            pl.BlockSpec((1, D1), lambda i: (0, 0)),
        ],
        out_shape=[
            jax.ShapeDtypeStruct((BS, D1), jnp.float32),
            jax.ShapeDtypeStruct((1, D1), jnp.float32),
            jax.ShapeDtypeStruct((1, D1), jnp.float32),
        ],
        scratch_shapes=[pltpu.VMEM((2, D1), jnp.float32)],
    )(h1, sm1, sq1, g1, be1, w2t, b2r)


def _k5_body(h_ref, sm_ref, sq_ref, g_ref, be_ref, o_ref):
    n = jnp.float32(BS)
    mean = sm_ref[...] / n
    var = sq_ref[...] / n - mean * mean
    scale = g_ref[...] * lax.rsqrt(var + EPS)
    o_ref[0] = jnp.transpose(
        jnp.maximum((h_ref[...] - mean) * scale + be_ref[...], 0.0))


def _bn2(h2, sm2, sq2, g2, be2):
    return pl.pallas_call(
        _k5_body,
        grid=(G3,),
        in_specs=[
            pl.BlockSpec((BS3, D1), lambda i: (i, 0)),
            pl.BlockSpec((1, D1), lambda i: (0, 0)),
            pl.BlockSpec((1, D1), lambda i: (0, 0)),
            pl.BlockSpec((1, D1), lambda i: (0, 0)),
            pl.BlockSpec((1, D1), lambda i: (0, 0)),
        ],
        out_specs=pl.BlockSpec((1, D1, BS3),
                               lambda i: (i // (S // BS3), 0, i % (S // BS3))),
        out_shape=jax.ShapeDtypeStruct((B, D1, S), jnp.float32),
    )(h2, sm2, sq2, g2, be2)


@jax.jit
def kernel(xyz1, xyz2, points1, points2, W1, b1, gamma1, beta1,
           W2, b2, gamma2, beta2):
    x2t = jnp.transpose(xyz2, (0, 2, 1))
    idx, w = _top3(xyz1, x2t)

    p1t = jnp.transpose(points1, (0, 2, 1)).reshape(BS // 4, D1)

    idxr = idx.reshape(BS, 3)
    wr = w.reshape(BS, 3)
    interp = _interp_sc(idxr[:, 0], idxr[:, 1], idxr[:, 2],
                        wr[:, 0], wr[:, 1], wr[:, 2], p1t)

    w1at = jnp.transpose(W1[:, :D2])
    w1bt = jnp.transpose(W1[:, D2:])
    h1, sm1, sq1 = _layer1(points2, interp, w1at, w1bt, b1.reshape(1, -1))
    h2, sm2, sq2 = _layer2(h1, sm1, sq1, gamma1.reshape(1, -1),
                           beta1.reshape(1, -1), jnp.transpose(W2),
                           b2.reshape(1, -1))
    return _bn2(h2, sm2, sq2, gamma2.reshape(1, -1), beta2.reshape(1, -1))


# speedup vs baseline: 1.1073x; 1.0619x over previous
import functools

import jax
import jax.numpy as jnp
from jax import lax
from jax.experimental import pallas as pl
from jax.experimental.pallas import tpu as pltpu
from jax.experimental.pallas import tpu_sc as plsc

B, N, S = 16, 1024, 4096
D1, D2 = 256, 256
BS = B * S
EPS = 1e-5

SB = 512


def _k1_body(x1_ref, x2_ref, idx_ref, w_ref):
    b = pl.program_id(0)
    x1 = x1_ref[0]
    x2 = x2_ref[0]
    x1a, x1b, x1c = x1[0:1, :], x1[1:2, :], x1[2:3, :]
    x2a, x2b, x2c = x2[:, 0:1], x2[:, 1:2], x2[:, 2:3]
    sq1 = x1a * x1a + x1b * x1b + x1c * x1c
    sq2 = x2a * x2a + x2b * x2b + x2c * x2c
    cross = lax.dot_general(x2, x1, (((1,), (0,)), ((), ())),
                            preferred_element_type=jnp.float32)
    d = -2.0 * cross + sq2 + sq1
    lane = lax.broadcasted_iota(jnp.int32, (SB, N), 1)
    inf = jnp.float32(jnp.inf)

    big = jnp.int32(1 << 30)
    m1 = jnp.min(d, axis=1, keepdims=True)
    e1 = d == m1
    d2 = jnp.where(e1, inf, d)
    m2 = jnp.min(d2, axis=1, keepdims=True)
    e2 = d2 == m2
    d3 = jnp.where(e2, inf, d2)
    m3 = jnp.min(d3, axis=1, keepdims=True)
    e3 = d3 == m3
    i1 = jnp.min(jnp.where(e1, lane, big), axis=1, keepdims=True)
    i2 = jnp.min(jnp.where(e2, lane, big), axis=1, keepdims=True)
    i3 = jnp.min(jnp.where(e3, lane, big), axis=1, keepdims=True)

    r1 = 1.0 / (m1 + 1e-8)
    r2 = 1.0 / (m2 + 1e-8)
    r3 = 1.0 / (m3 + 1e-8)
    nrm = r1 + r2 + r3
    idx_ref[0] = jnp.concatenate([i1, i2, i3], axis=1) + b * N
    w_ref[0] = jnp.concatenate([r1, r2, r3], axis=1) / nrm


def _top3(xyz1, x2t):
    return pl.pallas_call(
        _k1_body,
        grid=(B, S // SB),
        in_specs=[
            pl.BlockSpec((1, 3, N), lambda b, j: (b, 0, 0)),
            pl.BlockSpec((1, SB, 3), lambda b, j: (b, j, 0)),
        ],
        out_specs=[
            pl.BlockSpec((1, SB, 3), lambda b, j: (b, j, 0)),
            pl.BlockSpec((1, SB, 3), lambda b, j: (b, j, 0)),
        ],
        out_shape=[
            jax.ShapeDtypeStruct((B, S, 3), jnp.int32),
            jax.ShapeDtypeStruct((B, S, 3), jnp.float32),
        ],
    )(xyz1, x2t)


NW = 32
CH = BS // NW
T = 64
TS = CH // T


def _sc_body(idx0, idx1, idx2, w0, w1, w2, p1t, out_hbm,
             i0v, i1v, i2v, w0v, w1v, w2v, b0, b1, b2, ob, sem):
    c = lax.axis_index("c")
    s = lax.axis_index("s")
    wid = s * 2 + c
    base = pl.multiple_of(wid * CH, CH)
    pltpu.sync_copy(idx0.at[pl.ds(base, CH)], i0v)
    pltpu.sync_copy(idx1.at[pl.ds(base, CH)], i1v)
    pltpu.sync_copy(idx2.at[pl.ds(base, CH)], i2v)
    pltpu.sync_copy(w0.at[pl.ds(base, CH)], w0v)
    pltpu.sync_copy(w1.at[pl.ds(base, CH)], w1v)
    pltpu.sync_copy(w2.at[pl.ds(base, CH)], w2v)

    def step(t, carry):
        row0 = pl.multiple_of(t * T, T)
        cp0 = pltpu.async_copy(p1t.at[i0v.at[pl.ds(row0, T)]], b0, sem)
        cp1 = pltpu.async_copy(p1t.at[i1v.at[pl.ds(row0, T)]], b1, sem)
        cp2 = pltpu.async_copy(p1t.at[i2v.at[pl.ds(row0, T)]], b2, sem)
        cp0.wait()
        cp1.wait()
        cp2.wait()

        def grpfn(g, carry2):
            gbase = row0 + g * 16
            wv0 = w0v[pl.ds(gbase, 16)]
            wv1 = w1v[pl.ds(gbase, 16)]
            wv2 = w2v[pl.ds(gbase, 16)]
            for r16 in range(16):
                r = g * 16 + r16
                s0 = wv0[r16]
                s1 = wv1[r16]
                s2 = wv2[r16]
                for cc in range(D1 // 16):
                    sl = pl.ds(cc * 16, 16)
                    ob[r, sl] = (s0 * b0[r, sl] + s1 * b1[r, sl]
                                 + s2 * b2[r, sl])
            return carry2

        lax.fori_loop(0, T // 16, grpfn, 0)
        pltpu.sync_copy(ob, out_hbm.at[pl.ds(base + row0, T)])
        return carry

    lax.fori_loop(0, TS, step, 0)


def _interp_sc(idx0, idx1, idx2, w0, w1, w2, p1t):
    mesh = plsc.VectorSubcoreMesh(core_axis_name="c", subcore_axis_name="s")
    f = functools.partial(
        pl.kernel,
        mesh=mesh,
        out_type=jax.ShapeDtypeStruct((BS, D1), jnp.float32),
        scratch_types=[
            pltpu.VMEM((CH,), jnp.int32),
            pltpu.VMEM((CH,), jnp.int32),
            pltpu.VMEM((CH,), jnp.int32),
            pltpu.VMEM((CH,), jnp.float32),
            pltpu.VMEM((CH,), jnp.float32),
            pltpu.VMEM((CH,), jnp.float32),
            pltpu.VMEM((T, D1), jnp.float32),
            pltpu.VMEM((T, D1), jnp.float32),
            pltpu.VMEM((T, D1), jnp.float32),
            pltpu.VMEM((T, D1), jnp.float32),
            pltpu.SemaphoreType.DMA,
        ],
    )(_sc_body)
    return f(idx0, idx1, idx2, w0, w1, w2, p1t)


BS3 = 2048
G3 = BS // BS3


def _k3_body(p2_ref, it_ref, wa_ref, wb_ref, b1_ref, h_ref, sm_ref, sq_ref, acc):
    i = pl.program_id(0)

    @pl.when(i == 0)
    def _():
        acc[...] = jnp.zeros_like(acc)

    p2b = p2_ref[0]
    ib = it_ref[...]
    bf = jnp.bfloat16
    t1 = lax.dot_general(p2b.astype(bf), wa_ref[...].astype(bf),
                         (((0,), (0,)), ((), ())),
                         preferred_element_type=jnp.float32)
    t2 = lax.dot_general(ib.astype(bf), wb_ref[...].astype(bf),
                         (((1,), (0,)), ((), ())),
                         preferred_element_type=jnp.float32)
    h = t1 + t2 + b1_ref[...]
    h_ref[...] = h
    acc[0:1] += jnp.sum(h, axis=0, keepdims=True)
    acc[1:2] += jnp.sum(h * h, axis=0, keepdims=True)

    @pl.when(i == G3 - 1)
    def _():
        sm_ref[...] = acc[0:1]
        sq_ref[...] = acc[1:2]


def _layer1(p2, interp, w1at, w1bt, b1r):
    return pl.pallas_call(
        _k3_body,
        grid=(G3,),
        in_specs=[
            pl.BlockSpec((1, D2, BS3), lambda i: (i // (S // BS3), 0, i % (S // BS3))),
            pl.BlockSpec((BS3, D1), lambda i: (i, 0)),
            pl.BlockSpec((D2, D1), lambda i: (0, 0)),
            pl.BlockSpec((D1, D1), lambda i: (0, 0)),
            pl.BlockSpec((1, D1), lambda i: (0, 0)),
        ],
        out_specs=[
            pl.BlockSpec((BS3, D1), lambda i: (i, 0)),
            pl.BlockSpec((1, D1), lambda i: (0, 0)),
            pl.BlockSpec((1, D1), lambda i: (0, 0)),
        ],
        out_shape=[
            jax.ShapeDtypeStruct((BS, D1), jnp.float32),
            jax.ShapeDtypeStruct((1, D1), jnp.float32),
            jax.ShapeDtypeStruct((1, D1), jnp.float32),
        ],
        scratch_shapes=[pltpu.VMEM((2, D1), jnp.float32)],
    )(p2, interp, w1at, w1bt, b1r)


def _k4_body(h_ref, sm1_ref, sq1_ref, g1_ref, be1_ref, w2_ref, b2_ref,
             o_ref, sm_ref, sq_ref, acc):
    i = pl.program_id(0)

    @pl.when(i == 0)
    def _():
        acc[...] = jnp.zeros_like(acc)

    n = jnp.float32(BS)
    mean = sm1_ref[...] / n
    var = sq1_ref[...] / n - mean * mean
    scale = g1_ref[...] * lax.rsqrt(var + EPS)
    a = jnp.maximum((h_ref[...] - mean) * scale + be1_ref[...], 0.0)
    bf = jnp.bfloat16
    h2 = lax.dot_general(a.astype(bf), w2_ref[...].astype(bf),
                         (((1,), (0,)), ((), ())),
                         preferred_element_type=jnp.float32) + b2_ref[...]
    o_ref[...] = h2
    acc[0:1] += jnp.sum(h2, axis=0, keepdims=True)
    acc[1:2] += jnp.sum(h2 * h2, axis=0, keepdims=True)

    @pl.when(i == G3 - 1)
    def _():
        sm_ref[...] = acc[0:1]
        sq_ref[...] = acc[1:2]


def _layer2(h1, sm1, sq1, g1, be1, w2t, b2r):
    return pl.pallas_call(
        _k4_body,
        grid=(G3,),
        in_specs=[
            pl.BlockSpec((BS3, D1), lambda i: (i, 0)),
            pl.BlockSpec((1, D1), lambda i: (0, 0)),
            pl.BlockSpec((1, D1), lambda i: (0, 0)),
            pl.BlockSpec((1, D1), lambda i: (0, 0)),
            pl.BlockSpec((1, D1), lambda i: (0, 0)),
            pl.BlockSpec((D1, D1), lambda i: (0, 0)),
            pl.BlockSpec((1, D1), lambda i: (0, 0)),
        ],
        out_specs=[
            pl.BlockSpec((BS3, D1), lambda i: (i, 0)),
            pl.BlockSpec((1, D1), lambda i: (0, 0)),
            pl.BlockSpec((1, D1), lambda i: (0, 0)),
        ],
        out_shape=[
            jax.ShapeDtypeStruct((BS, D1), jnp.float32),
            jax.ShapeDtypeStruct((1, D1), jnp.float32),
            jax.ShapeDtypeStruct((1, D1), jnp.float32),
        ],
        scratch_shapes=[pltpu.VMEM((2, D1), jnp.float32)],
    )(h1, sm1, sq1, g1, be1, w2t, b2r)


def _k5_body(h_ref, sm_ref, sq_ref, g_ref, be_ref, o_ref):
    n = jnp.float32(BS)
    mean = sm_ref[...] / n
    var = sq_ref[...] / n - mean * mean
    scale = g_ref[...] * lax.rsqrt(var + EPS)
    o_ref[0] = jnp.transpose(
        jnp.maximum((h_ref[...] - mean) * scale + be_ref[...], 0.0))


def _bn2(h2, sm2, sq2, g2, be2):
    return pl.pallas_call(
        _k5_body,
        grid=(G3,),
        in_specs=[
            pl.BlockSpec((BS3, D1), lambda i: (i, 0)),
            pl.BlockSpec((1, D1), lambda i: (0, 0)),
            pl.BlockSpec((1, D1), lambda i: (0, 0)),
            pl.BlockSpec((1, D1), lambda i: (0, 0)),
            pl.BlockSpec((1, D1), lambda i: (0, 0)),
        ],
        out_specs=pl.BlockSpec((1, D1, BS3),
                               lambda i: (i // (S // BS3), 0, i % (S // BS3))),
        out_shape=jax.ShapeDtypeStruct((B, D1, S), jnp.float32),
    )(h2, sm2, sq2, g2, be2)


@jax.jit
def kernel(xyz1, xyz2, points1, points2, W1, b1, gamma1, beta1,
           W2, b2, gamma2, beta2):
    x2t = jnp.transpose(xyz2, (0, 2, 1))
    idx, w = _top3(xyz1, x2t)

    p1t = jnp.transpose(points1, (0, 2, 1)).reshape(BS // 4, D1)

    idxr = idx.reshape(BS, 3)
    wr = w.reshape(BS, 3)
    interp = _interp_sc(idxr[:, 0], idxr[:, 1], idxr[:, 2],
                        wr[:, 0], wr[:, 1], wr[:, 2], p1t)

    w1at = jnp.transpose(W1[:, :D2])
    w1bt = jnp.transpose(W1[:, D2:])
    h1, sm1, sq1 = _layer1(points2, interp, w1at, w1bt, b1.reshape(1, -1))
    h2, sm2, sq2 = _layer2(h1, sm1, sq1, gamma1.reshape(1, -1),
                           beta1.reshape(1, -1), jnp.transpose(W2),
                           b2.reshape(1, -1))
    return _bn2(h2, sm2, sq2, gamma2.reshape(1, -1), beta2.reshape(1, -1))
